# baseline (speedup 1.0000x reference)
import functools

import jax
import jax.numpy as jnp
from jax.experimental import pallas as pl
from jax.experimental.pallas import tpu as pltpu

_NSPLIT = 8


def _mm_body(*refs, mb, kb, k_total):
    x_refs = refs[:_NSPLIT]
    w_ref, b_ref, o_ref = refs[_NSPLIT:]
    k = pl.program_id(0)
    nk = pl.num_programs(0)

    @pl.when(k == 0)
    def _init():
        o_ref[...] = jnp.broadcast_to(b_ref[...], o_ref.shape)

    def _acc(w):
        for c, x_ref in enumerate(x_refs):
            x = x_ref[...]
            if w is None:
                wv = w_ref[...]
            else:
                xcol = jax.lax.broadcasted_iota(jnp.int32, x.shape, 1) + k * kb
                x = jnp.where(xcol < k_total, x, 0.0)
                wv = w
            o_ref[pl.ds(c * mb, mb), :] += jax.lax.dot_general(
                x, wv,
                dimension_numbers=(((1,), (0,)), ((), ())),
                preferred_element_type=jnp.float32)

    @pl.when(k < nk - 1)
    def _full():
        _acc(None)

    @pl.when(k == nk - 1)
    def _tail():
        w = w_ref[...]
        wrow = jax.lax.broadcasted_iota(jnp.int32, w.shape, 0) + k * kb
        _acc(jnp.where(wrow < k_total, w, 0.0))


def kernel(inputs, w, b):
    m, k_total = inputs.shape
    _, n = w.shape
    mb = m // _NSPLIT
    kb = 4096
    grid = (pl.cdiv(k_total, kb),)
    x_specs = [
        pl.BlockSpec((mb, kb), functools.partial(lambda c, j: (c, j), c))
        for c in range(_NSPLIT)
    ]
    out = pl.pallas_call(
        functools.partial(_mm_body, mb=mb, kb=kb, k_total=k_total),
        grid=grid,
        in_specs=x_specs + [
            pl.BlockSpec((kb, n), lambda j: (j, 0)),
            pl.BlockSpec((1, n), lambda j: (0, 0)),
        ],
        out_specs=pl.BlockSpec((m, n), lambda j: (0, 0)),
        out_shape=jax.ShapeDtypeStruct((m, n), jnp.float32),
        compiler_params=pltpu.CompilerParams(
            dimension_semantics=("arbitrary",),
        ),
    )(*([inputs] * _NSPLIT), w, b.reshape(1, n))
    return out


# speedup vs baseline: 3.9258x; 3.9258x over previous
import functools

import jax
import jax.numpy as jnp
from jax.experimental import pallas as pl
from jax.experimental.pallas import tpu as pltpu


def _mm_body(xt_ref, wt_ref, b_ref, o_ref, *, kb, k_total):
    k = pl.program_id(0)
    nk = pl.num_programs(0)

    @pl.when(k == 0)
    def _init():
        o_ref[...] = jnp.broadcast_to(b_ref[...], o_ref.shape)

    def _acc(xt, wt):
        o_ref[...] += jax.lax.dot_general(
            xt, wt,
            dimension_numbers=(((0,), (1,)), ((), ())),
            preferred_element_type=jnp.float32)

    @pl.when(k < nk - 1)
    def _full():
        _acc(xt_ref[...], wt_ref[...])

    @pl.when(k == nk - 1)
    def _tail():
        xt = xt_ref[...]
        wt = wt_ref[...]
        base = k * kb
        xrow = jax.lax.broadcasted_iota(jnp.int32, xt.shape, 0) + base
        wcol = jax.lax.broadcasted_iota(jnp.int32, wt.shape, 1) + base
        _acc(jnp.where(xrow < k_total, xt, 0.0),
             jnp.where(wcol < k_total, wt, 0.0))


def kernel(inputs, w, b):
    m, k_total = inputs.shape
    _, n = w.shape
    kb = 2048
    grid = (pl.cdiv(k_total, kb),)
    out = pl.pallas_call(
        functools.partial(_mm_body, kb=kb, k_total=k_total),
        grid=grid,
        in_specs=[
            pl.BlockSpec((kb, m), lambda j: (j, 0)),
            pl.BlockSpec((n, kb), lambda j: (0, j)),
            pl.BlockSpec((1, n), lambda j: (0, 0)),
        ],
        out_specs=pl.BlockSpec((m, n), lambda j: (0, 0)),
        out_shape=jax.ShapeDtypeStruct((m, n), jnp.float32),
        compiler_params=pltpu.CompilerParams(
            dimension_semantics=("arbitrary",),
        ),
    )(inputs.T, w.T, b.reshape(1, n))
    return out


# speedup vs baseline: 4.1794x; 1.0646x over previous
import functools

import jax
import jax.numpy as jnp
from jax.experimental import pallas as pl
from jax.experimental.pallas import tpu as pltpu


def _mm_body(xt_ref, wt_ref, b_ref, o_ref, *, kb, k_total):
    k = pl.program_id(0)
    nk = pl.num_programs(0)

    @pl.when(k == 0)
    def _init():
        o_ref[...] = jnp.broadcast_to(b_ref[...], o_ref.shape)

    def _acc(wt, xt):
        o_ref[...] += jax.lax.dot_general(
            wt, xt,
            dimension_numbers=(((1,), (0,)), ((), ())),
            preferred_element_type=jnp.float32)

    @pl.when(k < nk - 1)
    def _full():
        _acc(wt_ref[...], xt_ref[...])

    @pl.when(k == nk - 1)
    def _tail():
        xt = xt_ref[...]
        wt = wt_ref[...]
        base = k * kb
        xrow = jax.lax.broadcasted_iota(jnp.int32, xt.shape, 0) + base
        wcol = jax.lax.broadcasted_iota(jnp.int32, wt.shape, 1) + base
        _acc(jnp.where(wcol < k_total, wt, 0.0),
             jnp.where(xrow < k_total, xt, 0.0))


def kernel(inputs, w, b):
    m, k_total = inputs.shape
    _, n = w.shape
    kb = 2048
    grid = (pl.cdiv(k_total, kb),)
    out_t = pl.pallas_call(
        functools.partial(_mm_body, kb=kb, k_total=k_total),
        grid=grid,
        in_specs=[
            pl.BlockSpec((kb, m), lambda j: (j, 0)),
            pl.BlockSpec((n, kb), lambda j: (0, j)),
            pl.BlockSpec((n, 1), lambda j: (0, 0)),
        ],
        out_specs=pl.BlockSpec((n, m), lambda j: (0, 0)),
        out_shape=jax.ShapeDtypeStruct((n, m), jnp.float32),
        compiler_params=pltpu.CompilerParams(
            dimension_semantics=("arbitrary",),
        ),
    )(inputs.T, w.T, b.reshape(n, 1))
    return out_t.T
